# Initial kernel scaffold; baseline (speedup 1.0000x reference)
import jax
import jax.numpy as jnp
from jax.experimental import pallas as pl


def kernel(trainable, batch_vec):
    raise NotImplementedError("write your pallas kernel here")


# speedup vs baseline: 27.6706x; 27.6706x over previous
import jax
import jax.numpy as jnp
from jax.experimental import pallas as pl

_ROWS_PER_BLOCK = 4000


def _copy_block(trainable_ref, out_ref):
    out_ref[...] = trainable_ref[...]


def kernel(trainable, batch_vec):
    del batch_vec
    n, d = trainable.shape
    r = _ROWS_PER_BLOCK if n % _ROWS_PER_BLOCK == 0 else n
    grid = (n // r,)
    return pl.pallas_call(
        _copy_block,
        grid=grid,
        in_specs=[pl.BlockSpec((r, d), lambda i: (i, 0))],
        out_specs=pl.BlockSpec((r, d), lambda i: (i, 0)),
        out_shape=jax.ShapeDtypeStruct((n, d), jnp.float32),
    )(trainable)


# speedup vs baseline: 30.8924x; 1.1164x over previous
import jax
import jax.numpy as jnp
from jax.experimental import pallas as pl

_ROWS_PER_BLOCK = 10000


def _copy_block(trainable_ref, out_ref):
    out_ref[...] = trainable_ref[...]


def kernel(trainable, batch_vec):
    del batch_vec
    n, d = trainable.shape
    r = _ROWS_PER_BLOCK if n % _ROWS_PER_BLOCK == 0 else n
    grid = (n // r,)
    return pl.pallas_call(
        _copy_block,
        grid=grid,
        in_specs=[pl.BlockSpec((r, d), lambda i: (i, 0))],
        out_specs=pl.BlockSpec((r, d), lambda i: (i, 0)),
        out_shape=jax.ShapeDtypeStruct((n, d), jnp.float32),
    )(trainable)
